# baseline (speedup 1.0000x reference)
import functools

import jax
import jax.numpy as jnp
from jax import lax
from jax.experimental import pallas as pl
from jax.experimental.pallas import tpu as pltpu
from jax.experimental.pallas import tpu_sc as plsc

_K = 8192
_D = 32
_N = 8192
_RB = 256

_NC, _NS, _L = 2, 16, 16
_NW = _NC * _NS
_BPW = _N // _NW


_W = 2048


def _argmin_body(x_ref, cbt_ref, xsq_ref, csq_ref, col_ref, idx_ref):
    x = x_ref[...]
    cbt = cbt_ref[...]
    m2 = jnp.dot(x + x, cbt, preferred_element_type=jnp.float32)
    d = (xsq_ref[...] - m2) + csq_ref[...]
    nwin = _K // _W
    big = jnp.float32(2.0 * _K)
    mvs = [jnp.min(d[:, w * _W:(w + 1) * _W], axis=1) for w in range(nwin)]
    bv = mvs[0].astype(jnp.bfloat16).astype(jnp.float32)
    wsel = jnp.zeros(mvs[0].shape, jnp.float32)
    for w in range(1, nwin):
        take = mvs[w] < bv
        bv = jnp.where(take, mvs[w], bv).astype(jnp.bfloat16).astype(jnp.float32)
        wsel = jnp.where(take, jnp.float32(w), wsel)
    bi = None
    for w in range(nwin):
        target = jnp.where(wsel == jnp.float32(w), mvs[w], jnp.float32(jnp.nan))

        def _extract(w=w, target=target):
            dw = d[:, w * _W:(w + 1) * _W]
            colw = col_ref[:, w * _W:(w + 1) * _W]
            return jnp.min(jnp.where(dw == target[:, None], colw, big), axis=1)

        cand = lax.cond(jnp.any(wsel == jnp.float32(w)), _extract,
                        lambda: jnp.full(mvs[0].shape, big, jnp.float32))
        bi = cand if bi is None else jnp.minimum(bi, cand)
    idx_ref[...] = bi.astype(jnp.int32)


def _argmin_call(flat, cbt, xsq, csq, col):
    return pl.pallas_call(
        _argmin_body,
        grid=(_N // _RB,),
        in_specs=[
            pl.BlockSpec((_RB, _D), lambda i: (i, 0)),
            pl.BlockSpec((_D, _K), lambda i: (0, 0)),
            pl.BlockSpec((_RB, 1), lambda i: (i, 0)),
            pl.BlockSpec((1, _K), lambda i: (0, 0)),
            pl.BlockSpec((1, _K), lambda i: (0, 0)),
        ],
        out_specs=pl.BlockSpec((_RB,), lambda i: (i,)),
        out_shape=jax.ShapeDtypeStruct((_N,), jnp.int32),
    )(flat, cbt, xsq, csq, col)


def _sc_call(codebook, idx):
    mesh = plsc.VectorSubcoreMesh(core_axis_name="c", subcore_axis_name="s")

    @functools.partial(
        pl.kernel,
        mesh=mesh,
        out_type=[
            jax.ShapeDtypeStruct((_N, _D), jnp.float32),
            jax.ShapeDtypeStruct((_NW, _K), jnp.float32),
        ],
        scratch_types=[
            pltpu.VMEM((_BPW,), jnp.int32),
            pltpu.VMEM((_BPW, _D), jnp.float32),
            pltpu.VMEM((_K,), jnp.float32),
            pltpu.SemaphoreType.DMA,
        ],
        compiler_params=pltpu.CompilerParams(
            needs_layout_passes=False, use_tc_tiling_on_sc=False
        ),
    )
    def sc(cb_hbm, idx_hbm, codes_hbm, hist_hbm, idx_v, rows_v, hist_v, sem):
        wid = lax.axis_index("s") * _NC + lax.axis_index("c")
        base = wid * _BPW
        pltpu.sync_copy(idx_hbm.at[pl.ds(base, _BPW)], idx_v)
        pltpu.async_copy(cb_hbm.at[idx_v], rows_v, sem).wait()
        pltpu.sync_copy(rows_v, codes_hbm.at[pl.ds(base, _BPW)])

        zero16 = jnp.zeros((_L,), jnp.float32)

        def _zero(i, carry):
            hist_v[pl.ds(i * _L, _L)] = zero16
            return carry

        lax.fori_loop(0, _K // _L, _zero, 0)

        one16 = jnp.ones((_L,), jnp.float32)

        def _acc(j, carry):
            idx16 = idx_v[pl.ds(j * _L, _L)]
            plsc.addupdate_scatter(hist_v, [idx16], one16)
            return carry

        lax.fori_loop(0, _BPW // _L, _acc, 0)
        pltpu.sync_copy(hist_v, hist_hbm.at[wid])

    return sc(codebook, idx)


def _finalize_body(x_ref, c_ref, hist_ref, qst_ref, loss_ref, ppl_ref):
    x = x_ref[...]
    c = c_ref[...]
    qst_ref[...] = x + (c - x)
    diff = c - x
    loss_ref[0, 0] = 0.25 * (jnp.sum(diff * diff) / float(_N * _D))
    counts = jnp.sum(hist_ref[...], axis=0)
    p = counts * (1.0 / _N)
    ent = jnp.sum(p * jnp.log(p + 1e-10))
    ppl_ref[0, 0] = jnp.exp(-ent)


def _finalize_call(flat, codes, hist):
    return pl.pallas_call(
        _finalize_body,
        out_shape=[
            jax.ShapeDtypeStruct((_N, _D), jnp.float32),
            jax.ShapeDtypeStruct((1, 1), jnp.float32),
            jax.ShapeDtypeStruct((1, 1), jnp.float32),
        ],
        out_specs=[
            pl.BlockSpec(memory_space=pltpu.VMEM),
            pl.BlockSpec(memory_space=pltpu.SMEM),
            pl.BlockSpec(memory_space=pltpu.SMEM),
        ],
    )(flat, codes, hist)


def kernel(inputs, codebook):
    shape = inputs.shape
    flat = inputs.reshape(-1, shape[-1])
    cbt = codebook.T
    xsq = jnp.sum(flat**2, axis=-1, keepdims=True)
    csq = jnp.sum(codebook**2, axis=-1)[None, :]
    col = lax.iota(jnp.float32, _K)[None, :]
    idx = _argmin_call(flat, cbt, xsq, csq, col)
    codes, hist = _sc_call(codebook, idx)
    qst, loss, ppl = _finalize_call(flat, codes, hist)
    return qst.reshape(shape), loss[0, 0], ppl[0, 0]


# speedup vs baseline: 1.1282x; 1.1282x over previous
import functools

import jax
import jax.numpy as jnp
from jax import lax
from jax.experimental import pallas as pl
from jax.experimental.pallas import tpu as pltpu
from jax.experimental.pallas import tpu_sc as plsc

_K = 8192
_D = 32
_N = 8192
_RB = 512

_NC, _NS, _L = 2, 16, 16
_NW = _NC * _NS
_BPW = _N // _NW


_W = 2048


def _argmin_body(x_ref, cbt_ref, xsq_ref, csq_ref, col_ref, idx_ref):
    x = x_ref[...]
    cbt = cbt_ref[...]
    m2 = jnp.dot(x + x, cbt, preferred_element_type=jnp.float32)
    d = (xsq_ref[...] - m2) + csq_ref[...]
    bv = None
    for w in range(_K // _W):
        dw = d[:, w * _W:(w + 1) * _W]
        mv = jnp.min(dw, axis=1)
        col = col_ref[:, w * _W:(w + 1) * _W]
        iw = jnp.min(jnp.where(dw == mv[:, None], col, jnp.float32(2.0 * _K)), axis=1)
        if bv is None:
            bv = mv.astype(jnp.bfloat16).astype(jnp.float32)
            bi = iw
        else:
            take = (mv < bv) | ((mv == bv) & (iw < bi))
            bv = jnp.where(take, mv, bv).astype(jnp.bfloat16).astype(jnp.float32)
            bi = jnp.where(take, iw, bi)
    idx_ref[...] = bi.astype(jnp.int32)


def _argmin_call(flat, cbt, xsq, csq, col):
    return pl.pallas_call(
        _argmin_body,
        grid=(_N // _RB,),
        in_specs=[
            pl.BlockSpec((_RB, _D), lambda i: (i, 0)),
            pl.BlockSpec((_D, _K), lambda i: (0, 0)),
            pl.BlockSpec((_RB, 1), lambda i: (i, 0)),
            pl.BlockSpec((1, _K), lambda i: (0, 0)),
            pl.BlockSpec((1, _K), lambda i: (0, 0)),
        ],
        out_specs=pl.BlockSpec((_RB,), lambda i: (i,)),
        out_shape=jax.ShapeDtypeStruct((_N,), jnp.int32),
        compiler_params=pltpu.CompilerParams(
            allow_input_fusion=(True, True, True, True, True),
        ),
    )(flat, cbt, xsq, csq, col)


def _sc_call(codebook, idx):
    mesh = plsc.VectorSubcoreMesh(core_axis_name="c", subcore_axis_name="s")

    @functools.partial(
        pl.kernel,
        mesh=mesh,
        out_type=[
            jax.ShapeDtypeStruct((_N, _D), jnp.float32),
            jax.ShapeDtypeStruct((_NW, _K), jnp.float32),
        ],
        scratch_types=[
            pltpu.VMEM((_BPW,), jnp.int32),
            pltpu.VMEM((_BPW, _D), jnp.float32),
            pltpu.VMEM((_K,), jnp.float32),
            pltpu.SemaphoreType.DMA,
        ],
        compiler_params=pltpu.CompilerParams(
            needs_layout_passes=False, use_tc_tiling_on_sc=False
        ),
    )
    def sc(cb_hbm, idx_hbm, codes_hbm, hist_hbm, idx_v, rows_v, hist_v, sem):
        wid = lax.axis_index("s") * _NC + lax.axis_index("c")
        base = wid * _BPW
        pltpu.sync_copy(idx_hbm.at[pl.ds(base, _BPW)], idx_v)
        pltpu.async_copy(cb_hbm.at[idx_v], rows_v, sem).wait()
        pltpu.sync_copy(rows_v, codes_hbm.at[pl.ds(base, _BPW)])

        zero16 = jnp.zeros((_L,), jnp.float32)

        def _zero(i, carry):
            hist_v[pl.ds(i * _L, _L)] = zero16
            return carry

        lax.fori_loop(0, _K // _L, _zero, 0)

        one16 = jnp.ones((_L,), jnp.float32)

        def _acc(j, carry):
            idx16 = idx_v[pl.ds(j * _L, _L)]
            plsc.addupdate_scatter(hist_v, [idx16], one16)
            return carry

        lax.fori_loop(0, _BPW // _L, _acc, 0)
        pltpu.sync_copy(hist_v, hist_hbm.at[wid])

    return sc(codebook, idx)


def _finalize_body(x_ref, c_ref, hist_ref, qst_ref, loss_ref, ppl_ref):
    x = x_ref[...]
    c = c_ref[...]
    qst_ref[...] = x + (c - x)
    diff = c - x
    loss_ref[0, 0] = 0.25 * (jnp.sum(diff * diff) / float(_N * _D))
    counts = jnp.sum(hist_ref[...], axis=0)
    p = counts * (1.0 / _N)
    ent = jnp.sum(p * jnp.log(p + 1e-10))
    ppl_ref[0, 0] = jnp.exp(-ent)


def _finalize_call(flat, codes, hist):
    return pl.pallas_call(
        _finalize_body,
        out_shape=[
            jax.ShapeDtypeStruct((_N, _D), jnp.float32),
            jax.ShapeDtypeStruct((1, 1), jnp.float32),
            jax.ShapeDtypeStruct((1, 1), jnp.float32),
        ],
        out_specs=[
            pl.BlockSpec(memory_space=pltpu.VMEM),
            pl.BlockSpec(memory_space=pltpu.SMEM),
            pl.BlockSpec(memory_space=pltpu.SMEM),
        ],
    )(flat, codes, hist)


def kernel(inputs, codebook):
    shape = inputs.shape
    flat = inputs.reshape(-1, shape[-1])
    cbt = codebook.T
    xsq = jnp.sum(flat**2, axis=-1, keepdims=True)
    csq = jnp.sum(codebook**2, axis=-1)[None, :]
    col = lax.iota(jnp.float32, _K)[None, :]
    idx = _argmin_call(flat, cbt, xsq, csq, col)
    codes, hist = _sc_call(codebook, idx)
    qst, loss, ppl = _finalize_call(flat, codes, hist)
    return qst.reshape(shape), loss[0, 0], ppl[0, 0]


# speedup vs baseline: 1.1439x; 1.0139x over previous
import functools

import jax
import jax.numpy as jnp
from jax import lax
from jax.experimental import pallas as pl
from jax.experimental.pallas import tpu as pltpu
from jax.experimental.pallas import tpu_sc as plsc

_K = 8192
_D = 32
_N = 8192
_RB = 512

_NC, _NS, _L = 2, 16, 16
_NW = _NC * _NS
_BPW = _N // _NW


_W = 2048


def _argmin_body(x_ref, cbt_ref, xsq_ref, csq_ref, col_ref, idx_ref):
    x = x_ref[...]
    cbt = cbt_ref[...]
    m2 = jnp.dot(x + x, cbt, preferred_element_type=jnp.float32)
    d = (xsq_ref[...] - m2) + csq_ref[...]
    bv = None
    for w in range(_K // _W):
        dw = d[:, w * _W:(w + 1) * _W]
        mv = jnp.min(dw, axis=1)
        col = col_ref[:, w * _W:(w + 1) * _W]
        iw = jnp.min(jnp.where(dw == mv[:, None], col, jnp.float32(2.0 * _K)), axis=1)
        if bv is None:
            bv = mv.astype(jnp.bfloat16).astype(jnp.float32)
            bi = iw
        else:
            take = (mv < bv) | ((mv == bv) & (iw < bi))
            bv = jnp.where(take, mv, bv).astype(jnp.bfloat16).astype(jnp.float32)
            bi = jnp.where(take, iw, bi)
    idx_ref[...] = bi.astype(jnp.int32)


def _argmin_call(flat, cbt, xsq, csq, col):
    return pl.pallas_call(
        _argmin_body,
        grid=(_N // _RB,),
        in_specs=[
            pl.BlockSpec((_RB, _D), lambda i: (i, 0)),
            pl.BlockSpec((_D, _K), lambda i: (0, 0)),
            pl.BlockSpec((_RB, 1), lambda i: (i, 0)),
            pl.BlockSpec((1, _K), lambda i: (0, 0)),
            pl.BlockSpec((1, _K), lambda i: (0, 0)),
        ],
        out_specs=pl.BlockSpec((_RB,), lambda i: (i,)),
        out_shape=jax.ShapeDtypeStruct((_N,), jnp.int32),
    )(flat, cbt, xsq, csq, col)


def _sc_call(codebook, idx):
    mesh = plsc.VectorSubcoreMesh(core_axis_name="c", subcore_axis_name="s")

    @functools.partial(
        pl.kernel,
        mesh=mesh,
        out_type=[
            jax.ShapeDtypeStruct((_N, _D), jnp.float32),
            jax.ShapeDtypeStruct((_NW, _K), jnp.float32),
        ],
        scratch_types=[
            pltpu.VMEM((_BPW,), jnp.int32),
            pltpu.VMEM((_BPW, _D), jnp.float32),
            pltpu.VMEM((_K,), jnp.float32),
            pltpu.SemaphoreType.DMA,
        ],
        compiler_params=pltpu.CompilerParams(
            needs_layout_passes=False, use_tc_tiling_on_sc=False
        ),
    )
    def sc(cb_hbm, idx_hbm, codes_hbm, hist_hbm, idx_v, rows_v, hist_v, sem):
        wid = lax.axis_index("s") * _NC + lax.axis_index("c")
        base = wid * _BPW
        pltpu.sync_copy(idx_hbm.at[pl.ds(base, _BPW)], idx_v)
        pltpu.async_copy(cb_hbm.at[idx_v], rows_v, sem).wait()
        pltpu.sync_copy(rows_v, codes_hbm.at[pl.ds(base, _BPW)])

        zero16 = jnp.zeros((_L,), jnp.float32)

        def _zero(i, carry):
            hist_v[pl.ds(i * _L, _L)] = zero16
            return carry

        lax.fori_loop(0, _K // _L, _zero, 0)

        one16 = jnp.ones((_L,), jnp.float32)

        def _acc(j, carry):
            idx16 = idx_v[pl.ds(j * _L, _L)]
            plsc.addupdate_scatter(hist_v, [idx16], one16)
            return carry

        lax.fori_loop(0, _BPW // _L, _acc, 0)
        pltpu.sync_copy(hist_v, hist_hbm.at[wid])

    return sc(codebook, idx)


def _finalize_body(x_ref, c_ref, hist_ref, qst_ref, loss_ref, ppl_ref):
    x = x_ref[...]
    c = c_ref[...]
    qst_ref[...] = x + (c - x)
    diff = c - x
    loss_ref[0, 0] = 0.25 * (jnp.sum(diff * diff) / float(_N * _D))
    counts = jnp.sum(hist_ref[...], axis=0)
    p = counts * (1.0 / _N)
    ent = jnp.sum(p * jnp.log(p + 1e-10))
    ppl_ref[0, 0] = jnp.exp(-ent)


def _finalize_call(flat, codes, hist):
    return pl.pallas_call(
        _finalize_body,
        out_shape=[
            jax.ShapeDtypeStruct((_N, _D), jnp.float32),
            jax.ShapeDtypeStruct((1, 1), jnp.float32),
            jax.ShapeDtypeStruct((1, 1), jnp.float32),
        ],
        out_specs=[
            pl.BlockSpec(memory_space=pltpu.VMEM),
            pl.BlockSpec(memory_space=pltpu.SMEM),
            pl.BlockSpec(memory_space=pltpu.SMEM),
        ],
    )(flat, codes, hist)


def kernel(inputs, codebook):
    shape = inputs.shape
    flat = inputs.reshape(-1, shape[-1])
    cbt = codebook.T
    xsq = jnp.sum(flat**2, axis=-1, keepdims=True)
    csq = jnp.sum(codebook**2, axis=-1)[None, :]
    col = lax.iota(jnp.float32, _K)[None, :]
    idx = _argmin_call(flat, cbt, xsq, csq, col)
    codes, hist = _sc_call(codebook, idx)
    qst, loss, ppl = _finalize_call(flat, codes, hist)
    return qst.reshape(shape), loss[0, 0], ppl[0, 0]
